# Initial kernel scaffold; baseline (speedup 1.0000x reference)
import jax
import jax.numpy as jnp
from jax.experimental import pallas as pl


def kernel(x, edge_index, W_rel, b_rel, W_root, gamma, beta):
    raise NotImplementedError("write your pallas kernel here")


# speedup vs baseline: 3.2566x; 3.2566x over previous
import functools

import jax
import jax.numpy as jnp
from jax import lax
from jax.experimental import pallas as pl
from jax.experimental.pallas import tpu as pltpu
from jax.experimental.pallas import tpu_sc as plsc

N = 10000
E = 160000
D = 256
H = 128
EPS = 1e-5

NC = 2
NS = 16
BATCH = 128
NBATCH = 80
EPT = NBATCH * BATCH
E_PAD = NS * EPT
DUMMY = N
ACC_ROWS = NS * 5 * BATCH
ROWS_PER_TILE = ACC_ROWS // NS

_mesh = plsc.VectorSubcoreMesh(core_axis_name="c", subcore_axis_name="s")


@functools.partial(
    pl.kernel,
    out_type=[
        jax.ShapeDtypeStruct((ACC_ROWS, H), jnp.float32),
        jax.ShapeDtypeStruct((ACC_ROWS, H), jnp.float32),
    ],
    mesh=_mesh,
    scratch_types=[
        pltpu.VMEM((NBATCH, BATCH), jnp.int32),
        pltpu.VMEM((NBATCH, BATCH), jnp.int32),
        pltpu.VMEM((BATCH, H), jnp.float32),
        pltpu.VMEM_SHARED((ACC_ROWS, H), jnp.float32),
        pltpu.SemaphoreType.DMA,
    ],
)
def _sc_segment_sum(x0_hbm, x1_hbm, src_hbm, dst_hbm, agg0_hbm, agg1_hbm,
                    srcb, dstb, rows, acc, sem):
    cid = lax.axis_index("c")
    sid = lax.axis_index("s")

    def _zero_row(i, carry):
        for k in range(H // 16):
            rows[i, pl.ds(k * 16, 16)] = jnp.zeros((16,), jnp.float32)
        return carry

    lax.fori_loop(0, BATCH, _zero_row, 0)
    for j in range(ROWS_PER_TILE // BATCH):
        pltpu.sync_copy(rows, acc.at[pl.ds(sid * ROWS_PER_TILE + j * BATCH, BATCH)])

    pltpu.sync_copy(src_hbm.at[sid], srcb)
    pltpu.sync_copy(dst_hbm.at[sid], dstb)
    plsc.subcore_barrier()

    def _process(x_hbm):
        def step(j, carry):
            pltpu.async_copy(x_hbm.at[srcb.at[j]], rows, sem).wait()
            pltpu.sync_copy(rows, acc.at[dstb.at[j]], add=True)
            return carry
        lax.fori_loop(0, NBATCH, step, 0)

    @pl.when(cid == 0)
    def _():
        _process(x0_hbm)

    @pl.when(cid == 1)
    def _():
        _process(x1_hbm)

    plsc.subcore_barrier()

    def _copy_out(agg_hbm):
        for k in range(ROWS_PER_TILE // BATCH):
            r0 = sid * ROWS_PER_TILE + k * BATCH
            pltpu.sync_copy(acc.at[pl.ds(r0, BATCH)], rows)
            pltpu.sync_copy(rows, agg_hbm.at[pl.ds(r0, BATCH)])

    @pl.when(cid == 0)
    def _():
        _copy_out(agg0_hbm)

    @pl.when(cid == 1)
    def _():
        _copy_out(agg1_hbm)


BLK = 400
NBLK = N // BLK


def _tc_body(agg0_ref, agg1_ref, x_ref, wr0_ref, wr1_ref, wroot_ref,
             brel_ref, gamma_ref, beta_ref, out_ref, obuf, ssum, ssq, scl, sft):
    p = pl.program_id(0)
    i = pl.program_id(1)

    @pl.when(p == 0)
    def _():
        o = (jnp.dot(agg0_ref[...], wr0_ref[...], preferred_element_type=jnp.float32)
             + jnp.dot(agg1_ref[...], wr1_ref[...], preferred_element_type=jnp.float32)
             + jnp.dot(x_ref[...], wroot_ref[...], preferred_element_type=jnp.float32)
             + brel_ref[...])
        obuf[pl.ds(i * BLK, BLK), :] = o
        s = jnp.sum(o, axis=0, keepdims=True)
        q = jnp.sum(o * o, axis=0, keepdims=True)

        @pl.when(i == 0)
        def _():
            ssum[...] = s
            ssq[...] = q

        @pl.when(i > 0)
        def _():
            ssum[...] += s
            ssq[...] += q

    @pl.when(p == 1)
    def _():
        @pl.when(i == 0)
        def _():
            mean = ssum[...] / N
            var = ssq[...] / N - mean * mean
            g = gamma_ref[...] * lax.rsqrt(var + EPS)
            scl[...] = g
            sft[...] = beta_ref[...] - mean * g

        o = obuf[pl.ds(i * BLK, BLK), :]
        out_ref[...] = jnp.maximum(o * scl[...] + sft[...], 0.0)


def _tc_fused(agg0, agg1, x, wr0, wr1, wroot, brel, gamma, beta):
    row_spec = lambda w: pl.BlockSpec((BLK, w), lambda p, i: (jnp.where(p == 0, i, 0), 0))
    full_spec = lambda s: pl.BlockSpec(s, lambda p, i: (0, 0))
    return pl.pallas_call(
        _tc_body,
        grid=(2, NBLK),
        in_specs=[
            row_spec(H), row_spec(H), row_spec(D),
            full_spec((H, D)), full_spec((H, D)), full_spec((D, D)),
            full_spec((1, D)), full_spec((1, D)), full_spec((1, D)),
        ],
        out_specs=pl.BlockSpec((BLK, D), lambda p, i: (jnp.where(p == 1, i, 0), 0)),
        out_shape=jax.ShapeDtypeStruct((N, D), jnp.float32),
        scratch_shapes=[
            pltpu.VMEM((N, D), jnp.float32),
            pltpu.VMEM((1, D), jnp.float32),
            pltpu.VMEM((1, D), jnp.float32),
            pltpu.VMEM((1, D), jnp.float32),
            pltpu.VMEM((1, D), jnp.float32),
        ],
    )(agg0, agg1, x, wr0, wr1, wroot, brel, gamma, beta)


def kernel(x, edge_index, W_rel, b_rel, W_root, gamma, beta):
    src = edge_index[0].astype(jnp.int32)
    dst = edge_index[1].astype(jnp.int32)
    pad = E_PAD - E
    src_p = jnp.concatenate([src, jnp.zeros((pad,), jnp.int32)]).reshape(NS, NBATCH, BATCH)
    dst_p = jnp.concatenate([dst, jnp.full((pad,), DUMMY, jnp.int32)]).reshape(NS, NBATCH, BATCH)
    x0 = x[:, :H]
    x1 = x[:, H:]
    agg0, agg1 = _sc_segment_sum(x0, x1, src_p, dst_p)
    agg0 = agg0[:N]
    agg1 = agg1[:N]
    wrT = W_rel.T
    return _tc_fused(agg0, agg1, x, wrT[:H], wrT[H:], W_root.T,
                     b_rel.reshape(1, D), gamma.reshape(1, D), beta.reshape(1, D))
